# speedup vs baseline: 1.1900x; 1.1900x over previous
import functools

import jax
import jax.numpy as jnp
from jax import lax
from jax.experimental import pallas as pl
from jax.experimental.pallas import tpu as pltpu

N, E, H = 10000, 320000, 128
BE = 3200
BN = 2000


def _ln(h, g, b):
    mu = jnp.mean(h, axis=-1, keepdims=True)
    v = jnp.mean((h - mu) ** 2, axis=-1, keepdims=True)
    return (h - mu) * lax.rsqrt(v + 1e-5) * g + b


def _mm(a, w):
    return jnp.dot(a, w, preferred_element_type=jnp.float32)


def _node_enc_body(x_ref, mean_ref, std_ref, w0_ref, b0_ref, w1_ref, b1_ref,
                   g_ref, be_ref, o_ref):
    xn = (x_ref[...] - mean_ref[...]) / std_ref[...]
    h = jax.nn.relu(_mm(xn, w0_ref[...]) + b0_ref[...])
    h = _mm(h, w1_ref[...]) + b1_ref[...]
    o_ref[...] = _ln(h, g_ref[...], be_ref[...])


def _edge_enc_body(ea_ref, mean_ref, std_ref, w0_ref, b0_ref, w1_ref, b1_ref,
                   g_ref, be_ref, o_ref):
    ean = (ea_ref[...] - mean_ref[...]) / std_ref[...]
    h = jax.nn.relu(_mm(ean, w0_ref[...]) + b0_ref[...])
    h = _mm(h, w1_ref[...]) + b1_ref[...]
    o_ref[...] = _ln(h, g_ref[...], be_ref[...])


def _edge_mlp_body(xi_ref, xj_ref, ea_ref, wa_ref, wb_ref, wc_ref, b0_ref,
                   w1_ref, b1_ref, g_ref, be_ref, o_ref):
    ea = ea_ref[...]
    h = (_mm(xi_ref[...], wa_ref[...]) + _mm(xj_ref[...], wb_ref[...])
         + _mm(ea, wc_ref[...]) + b0_ref[...])
    h = jax.nn.relu(h)
    h = _mm(h, w1_ref[...]) + b1_ref[...]
    o_ref[...] = _ln(h, g_ref[...], be_ref[...]) + ea


def _node_mlp_body(x_ref, p0_ref, p1_ref, wa_ref, wb_ref, b0_ref,
                   w1_ref, b1_ref, g_ref, be_ref, o_ref):
    x = x_ref[...]
    agg = p0_ref[...] + p1_ref[...]
    h = _mm(x, wa_ref[...]) + _mm(agg, wb_ref[...]) + b0_ref[...]
    h = jax.nn.relu(h)
    h = _mm(h, w1_ref[...]) + b1_ref[...]
    o_ref[...] = x + _ln(h, g_ref[...], be_ref[...])


def _dec_body(x_ref, w0_ref, b0_ref, w1_ref, b1_ref, o_ref):
    h = jax.nn.relu(_mm(x_ref[...], w0_ref[...]) + b0_ref[...])
    o_ref[...] = _mm(h, w1_ref[...]) + b1_ref[...]


def _full(shape):
    nd = len(shape)
    return pl.BlockSpec(shape, lambda i: (0,) * nd)


def _rows(b, d):
    return pl.BlockSpec((b, d), lambda i: (i, 0))


def _tc_call(body, nrows, brows, in_specs, out_dim, dtype=jnp.float32):
    return pl.pallas_call(
        body,
        grid=(nrows // brows,),
        in_specs=in_specs,
        out_specs=_rows(brows, out_dim),
        out_shape=jax.ShapeDtypeStruct((nrows, out_dim), dtype),
        compiler_params=pltpu.CompilerParams(
            dimension_semantics=("arbitrary",)),
    )


def _node_enc(x, mean_x, std_x, p):
    specs = [_rows(BN, 128), _full((128,)), _full((128,)),
             _full((128, H)), _full((H,)), _full((H, H)), _full((H,)),
             _full((H,)), _full((H,))]
    return _tc_call(_node_enc_body, N, BN, specs, H)(
        x, mean_x, std_x, p['w0'], p['b0'], p['w1'], p['b1'], p['g'], p['be'])


def _edge_enc(ea, mean_e, std_e, p):
    specs = [_rows(BE, 4), _full((4,)), _full((4,)),
             _full((4, H)), _full((H,)), _full((H, H)), _full((H,)),
             _full((H,)), _full((H,))]
    return _tc_call(_edge_enc_body, E, BE, specs, H)(
        ea, mean_e, std_e, p['w0'], p['b0'], p['w1'], p['b1'], p['g'], p['be'])


def _edge_mlp(xi, xj, ea, wa, wb, wc, p):
    specs = [_rows(BE, H), _rows(BE, H), _rows(BE, H),
             _full((H, H)), _full((H, H)), _full((H, H)), _full((H,)),
             _full((H, H)), _full((H,)), _full((H,)), _full((H,))]
    return _tc_call(_edge_mlp_body, E, BE, specs, H)(
        xi, xj, ea, wa, wb, wc, p['b0'], p['w1'], p['b1'], p['g'], p['be'])


def _node_mlp(x, p0, p1, wa, wb, p):
    specs = [_rows(BN, H), _rows(BN, H), _rows(BN, H),
             _full((H, H)), _full((H, H)), _full((H,)),
             _full((H, H)), _full((H,)), _full((H,)), _full((H,))]
    return _tc_call(_node_mlp_body, N, BN, specs, H)(
        x, p0, p1, wa, wb, p['b0'], p['w1'], p['b1'], p['g'], p['be'])


def _decoder(x, p):
    w1p = jnp.zeros((H, 8), jnp.float32).at[:, :3].set(p['w1'])
    b1p = jnp.zeros((8,), jnp.float32).at[:3].set(p['b1'])
    specs = [_rows(BN, H), _full((H, H)), _full((H,)),
             _full((H, 8)), _full((8,))]
    out = _tc_call(_dec_body, N, BN, specs, 8)(
        x, p['w0'], p['b0'], w1p, b1p)
    return out[:, :3]


def kernel(x, edge_index, edge_attr, mean_x, std_x, mean_edge, std_edge,
           params):
    src = edge_index[0]
    dst = edge_index[1]

    x = _node_enc(x, mean_x, std_x, params['node_enc'])
    ea = _edge_enc(edge_attr, mean_edge, std_edge, params['edge_enc'])

    for lp in params['layers']:
        ew = lp['edge']
        wa, wb, wc = ew['w0'][:H], ew['w0'][H:2 * H], ew['w0'][2 * H:]
        xi = jnp.take(x, dst, axis=0)
        xj = jnp.take(x, src, axis=0)
        ue = _edge_mlp(xi, xj, ea, wa, wb, wc, ew)
        agg = jax.ops.segment_sum(ue, src, num_segments=N)
        nw = lp['node']
        x = _node_mlp(x, agg, jnp.zeros_like(agg), nw['w0'][:H],
                      nw['w0'][H:], nw)
        ea = ue

    return _decoder(x, params['dec'])


# speedup vs baseline: 3.1196x; 2.6216x over previous
import functools

import jax
import jax.numpy as jnp
from jax import lax
from jax.experimental import pallas as pl
from jax.experimental.pallas import tpu as pltpu
from jax.experimental.pallas import tpu_sc as plsc

N, E, H = 10000, 320000, 128
BE = 3200
BN = 2000

NC, NS = 2, 16
NW = NC * NS
EPW = E // NW
CH = 80
NCHUNK = EPW // CH
NPAD = 10240
NSLAB = NPAD // NS


def _sc_mesh():
    return plsc.VectorSubcoreMesh(core_axis_name="c", subcore_axis_name="s")


def _sc_gather2(x, dst, src):
    def body(x_hbm, dst_hbm, src_hbm, xi_hbm, xj_hbm,
             idx1, idx2, buf1, buf2, sem1, sem2):
        c = lax.axis_index("c")
        s = lax.axis_index("s")
        base0 = (s * NC + c) * EPW

        def step(i, carry):
            base = base0 + i * CH
            pltpu.sync_copy(dst_hbm.at[pl.ds(base, CH)], idx1)
            pltpu.sync_copy(src_hbm.at[pl.ds(base, CH)], idx2)
            cp1 = pltpu.async_copy(x_hbm.at[idx1], buf1, sem1)
            cp2 = pltpu.async_copy(x_hbm.at[idx2], buf2, sem2)
            cp1.wait()
            cp2.wait()
            pltpu.sync_copy(buf1, xi_hbm.at[pl.ds(base, CH)])
            pltpu.sync_copy(buf2, xj_hbm.at[pl.ds(base, CH)])
            return carry

        lax.fori_loop(0, NCHUNK, step, 0)

    f = pl.kernel(
        body,
        out_type=[jax.ShapeDtypeStruct((E, H), jnp.float32),
                  jax.ShapeDtypeStruct((E, H), jnp.float32)],
        mesh=_sc_mesh(),
        scratch_types=[pltpu.VMEM((CH,), jnp.int32),
                       pltpu.VMEM((CH,), jnp.int32),
                       pltpu.VMEM((CH, H), jnp.float32),
                       pltpu.VMEM((CH, H), jnp.float32),
                       pltpu.SemaphoreType.DMA,
                       pltpu.SemaphoreType.DMA],
        name="sc_gather2",
    )
    return f(x, dst, src)


def _sc_scatter_add(ue, src, zslab):
    def body(ue_hbm, src_hbm, z_hbm, out_hbm, rows_v, idx_v, acc):
        c = lax.axis_index("c")
        s = lax.axis_index("s")
        pltpu.sync_copy(z_hbm, acc.at[pl.ds(s * NSLAB, NSLAB)])
        plsc.subcore_barrier()
        base0 = c * (E // NC) + s * EPW

        def step(i, carry):
            base = base0 + i * CH
            pltpu.sync_copy(src_hbm.at[pl.ds(base, CH)], idx_v)
            pltpu.sync_copy(ue_hbm.at[pl.ds(base, CH)], rows_v)
            pltpu.sync_copy(rows_v, acc.at[idx_v], add=True)
            return carry

        lax.fori_loop(0, NCHUNK, step, 0)
        plsc.subcore_barrier()
        pltpu.sync_copy(acc.at[pl.ds(s * NSLAB, NSLAB)],
                        out_hbm.at[c, pl.ds(s * NSLAB, NSLAB)])

    f = pl.kernel(
        body,
        out_type=jax.ShapeDtypeStruct((NC, NPAD, H), jnp.float32),
        mesh=_sc_mesh(),
        scratch_types=[pltpu.VMEM((CH, H), jnp.float32),
                       pltpu.VMEM((CH,), jnp.int32),
                       pltpu.VMEM_SHARED((NPAD, H), jnp.float32)],
        name="sc_scatter_add",
    )
    return f(ue, src, zslab)


def _ln(h, g, b):
    mu = jnp.mean(h, axis=-1, keepdims=True)
    v = jnp.mean((h - mu) ** 2, axis=-1, keepdims=True)
    return (h - mu) * lax.rsqrt(v + 1e-5) * g + b


def _mm(a, w):
    return jnp.dot(a, w, preferred_element_type=jnp.float32)


def _node_enc_body(x_ref, mean_ref, std_ref, w0_ref, b0_ref, w1_ref, b1_ref,
                   g_ref, be_ref, o_ref):
    xn = (x_ref[...] - mean_ref[...]) / std_ref[...]
    h = jax.nn.relu(_mm(xn, w0_ref[...]) + b0_ref[...])
    h = _mm(h, w1_ref[...]) + b1_ref[...]
    o_ref[...] = _ln(h, g_ref[...], be_ref[...])


def _edge_enc_body(ea_ref, mean_ref, std_ref, w0_ref, b0_ref, w1_ref, b1_ref,
                   g_ref, be_ref, o_ref):
    ean = (ea_ref[...] - mean_ref[...]) / std_ref[...]
    h = jax.nn.relu(_mm(ean, w0_ref[...]) + b0_ref[...])
    h = _mm(h, w1_ref[...]) + b1_ref[...]
    o_ref[...] = _ln(h, g_ref[...], be_ref[...])


def _edge_mlp_body(xi_ref, xj_ref, ea_ref, wa_ref, wb_ref, wc_ref, b0_ref,
                   w1_ref, b1_ref, g_ref, be_ref, o_ref):
    ea = ea_ref[...]
    h = (_mm(xi_ref[...], wa_ref[...]) + _mm(xj_ref[...], wb_ref[...])
         + _mm(ea, wc_ref[...]) + b0_ref[...])
    h = jax.nn.relu(h)
    h = _mm(h, w1_ref[...]) + b1_ref[...]
    o_ref[...] = _ln(h, g_ref[...], be_ref[...]) + ea


def _node_mlp_body(x_ref, p0_ref, p1_ref, wa_ref, wb_ref, b0_ref,
                   w1_ref, b1_ref, g_ref, be_ref, o_ref):
    x = x_ref[...]
    agg = p0_ref[...] + p1_ref[...]
    h = _mm(x, wa_ref[...]) + _mm(agg, wb_ref[...]) + b0_ref[...]
    h = jax.nn.relu(h)
    h = _mm(h, w1_ref[...]) + b1_ref[...]
    o_ref[...] = x + _ln(h, g_ref[...], be_ref[...])


def _dec_body(x_ref, w0_ref, b0_ref, w1_ref, b1_ref, o_ref):
    h = jax.nn.relu(_mm(x_ref[...], w0_ref[...]) + b0_ref[...])
    o_ref[...] = _mm(h, w1_ref[...]) + b1_ref[...]


def _full(shape):
    nd = len(shape)
    return pl.BlockSpec(shape, lambda i: (0,) * nd)


def _rows(b, d):
    return pl.BlockSpec((b, d), lambda i: (i, 0))


def _tc_call(body, nrows, brows, in_specs, out_dim, dtype=jnp.float32):
    return pl.pallas_call(
        body,
        grid=(nrows // brows,),
        in_specs=in_specs,
        out_specs=_rows(brows, out_dim),
        out_shape=jax.ShapeDtypeStruct((nrows, out_dim), dtype),
        compiler_params=pltpu.CompilerParams(
            dimension_semantics=("arbitrary",)),
    )


def _node_enc(x, mean_x, std_x, p):
    specs = [_rows(BN, 128), _full((128,)), _full((128,)),
             _full((128, H)), _full((H,)), _full((H, H)), _full((H,)),
             _full((H,)), _full((H,))]
    return _tc_call(_node_enc_body, N, BN, specs, H)(
        x, mean_x, std_x, p['w0'], p['b0'], p['w1'], p['b1'], p['g'], p['be'])


def _edge_enc(ea, mean_e, std_e, p):
    specs = [_rows(BE, 4), _full((4,)), _full((4,)),
             _full((4, H)), _full((H,)), _full((H, H)), _full((H,)),
             _full((H,)), _full((H,))]
    return _tc_call(_edge_enc_body, E, BE, specs, H)(
        ea, mean_e, std_e, p['w0'], p['b0'], p['w1'], p['b1'], p['g'], p['be'])


def _edge_mlp(xi, xj, ea, wa, wb, wc, p):
    specs = [_rows(BE, H), _rows(BE, H), _rows(BE, H),
             _full((H, H)), _full((H, H)), _full((H, H)), _full((H,)),
             _full((H, H)), _full((H,)), _full((H,)), _full((H,))]
    return _tc_call(_edge_mlp_body, E, BE, specs, H)(
        xi, xj, ea, wa, wb, wc, p['b0'], p['w1'], p['b1'], p['g'], p['be'])


def _node_mlp(x, p0, p1, wa, wb, p):
    specs = [_rows(BN, H), _rows(BN, H), _rows(BN, H),
             _full((H, H)), _full((H, H)), _full((H,)),
             _full((H, H)), _full((H,)), _full((H,)), _full((H,))]
    return _tc_call(_node_mlp_body, N, BN, specs, H)(
        x, p0, p1, wa, wb, p['b0'], p['w1'], p['b1'], p['g'], p['be'])


def _decoder(x, p):
    w1p = jnp.zeros((H, 8), jnp.float32).at[:, :3].set(p['w1'])
    b1p = jnp.zeros((8,), jnp.float32).at[:3].set(p['b1'])
    specs = [_rows(BN, H), _full((H, H)), _full((H,)),
             _full((H, 8)), _full((8,))]
    out = _tc_call(_dec_body, N, BN, specs, 8)(
        x, p['w0'], p['b0'], w1p, b1p)
    return out[:, :3]


def kernel(x, edge_index, edge_attr, mean_x, std_x, mean_edge, std_edge,
           params):
    src = edge_index[0]
    dst = edge_index[1]

    x = _node_enc(x, mean_x, std_x, params['node_enc'])
    ea = _edge_enc(edge_attr, mean_edge, std_edge, params['edge_enc'])

    zslab = jnp.zeros((NSLAB, H), jnp.float32)
    for lp in params['layers']:
        ew = lp['edge']
        wa, wb, wc = ew['w0'][:H], ew['w0'][H:2 * H], ew['w0'][2 * H:]
        xi, xj = _sc_gather2(x, dst, src)
        ue = _edge_mlp(xi, xj, ea, wa, wb, wc, ew)
        part = _sc_scatter_add(ue, src, zslab)
        nw = lp['node']
        x = _node_mlp(x, part[0, :N], part[1, :N], nw['w0'][:H],
                      nw['w0'][H:], nw)
        ea = ue

    return _decoder(x, params['dec'])


# speedup vs baseline: 4.3231x; 1.3858x over previous
import jax
import jax.numpy as jnp
from jax import lax
from jax.experimental import pallas as pl
from jax.experimental.pallas import tpu as pltpu
from jax.experimental.pallas import tpu_sc as plsc

N, E, H = 10000, 320000, 128
HH = H // 2
BE = 3200
BN = 2048

NC, NS = 2, 16
NPAD = 10240
NSLAB = NPAD // NS

NB = 5
CH = 80
EPW = E // NS
NCH = EPW // CH
NMAC = NCH // NB

NW = NC * NS
EPWS = E // NW
CHS = 40
NCHS = EPWS // CHS
NMACS = NCHS // NB


def _ln(h, g, b):
    mu = jnp.mean(h, axis=-1, keepdims=True)
    v = jnp.mean((h - mu) ** 2, axis=-1, keepdims=True)
    return (h - mu) * lax.rsqrt(v + 1e-5) * g + b


def _mm(a, w):
    return jnp.dot(a, w, preferred_element_type=jnp.float32)


def _sc_mesh():
    return plsc.VectorSubcoreMesh(core_axis_name="c", subcore_axis_name="s")


def _gather_pipe(x_hbm, idx_hbm, out_hbm, ibufs, bufs, isem, gs, s):
    base0 = s * EPW

    def mac(g, carry):
        ips, cps = [], []
        for b in range(NB):
            q = g * NB + b
            ips.append(pltpu.async_copy(idx_hbm.at[s, q], ibufs[b],
                                        isem[b]))
        for b in range(NB):
            ips[b].wait()
            cps.append(pltpu.async_copy(x_hbm.at[ibufs[b]], bufs[b],
                                        gs[b]))
        for b in range(NB):
            q = g * NB + b
            base = base0 + q * CH
            cps[b].wait()
            pltpu.sync_copy(bufs[b], out_hbm.at[pl.ds(base, CH)])
        return carry

    lax.fori_loop(0, NMAC, mac, 0)


def _sc_gather2(x, dst2, src2):
    def body(x_hbm, dst_hbm, src_hbm, xi_hbm, xj_hbm, *rest):
        ibufs = rest[0:NB]
        bufs = rest[NB:2 * NB]
        isem = rest[2 * NB:3 * NB]
        gs = rest[3 * NB:4 * NB]
        c = lax.axis_index("c")
        s = lax.axis_index("s")

        @pl.when(c == 0)
        def _xi():
            _gather_pipe(x_hbm, dst_hbm, xi_hbm, ibufs, bufs, isem, gs, s)

        @pl.when(c == 1)
        def _xj():
            _gather_pipe(x_hbm, src_hbm, xj_hbm, ibufs, bufs, isem, gs, s)

    scratch = ([pltpu.VMEM((CH,), jnp.int32)] * NB
               + [pltpu.VMEM((CH, H), jnp.float32)] * NB
               + [pltpu.SemaphoreType.DMA] * (2 * NB))
    f = pl.kernel(
        body,
        out_type=[jax.ShapeDtypeStruct((E, H), jnp.float32),
                  jax.ShapeDtypeStruct((E, H), jnp.float32)],
        mesh=_sc_mesh(),
        scratch_types=scratch,
        name="sc_gather2",
    )
    return f(x, dst2, src2)


def _sc_scatter_add(ue, srcs, zslab):
    def body(ue_hbm, src_hbm, z_hbm, out_hbm, *rest):
        ibufs = rest[0:NB]
        bufs = rest[NB:2 * NB]
        isem = rest[2 * NB:3 * NB]
        lsem = rest[3 * NB:4 * NB]
        acc = rest[4 * NB]
        c = lax.axis_index("c")
        s = lax.axis_index("s")
        w = s * NC + c
        base0 = w * EPWS
        pltpu.sync_copy(z_hbm, acc.at[pl.ds(s * NSLAB, NSLAB)])
        plsc.subcore_barrier()

        def mac(g, carry):
            ips, cps = [], []
            for b in range(NB):
                q = g * NB + b
                base = base0 + q * CHS
                ips.append(pltpu.async_copy(src_hbm.at[w, q], ibufs[b],
                                            isem[b]))
                cps.append(pltpu.async_copy(ue_hbm.at[pl.ds(base, CHS)],
                                            bufs[b], lsem[b]))
            for b in range(NB):
                ips[b].wait()
                cps[b].wait()
                pltpu.sync_copy(bufs[b], acc.at[ibufs[b]], add=True)
            return carry

        lax.fori_loop(0, NMACS, mac, 0)
        plsc.subcore_barrier()
        pltpu.sync_copy(acc.at[pl.ds(s * NSLAB, NSLAB)],
                        out_hbm.at[c, pl.ds(s * NSLAB, NSLAB)])

    scratch = ([pltpu.VMEM((CHS,), jnp.int32)] * NB
               + [pltpu.VMEM((CHS, H), jnp.float32)] * NB
               + [pltpu.SemaphoreType.DMA] * (2 * NB)
               + [pltpu.VMEM_SHARED((NPAD, H), jnp.float32)])
    f = pl.kernel(
        body,
        out_type=jax.ShapeDtypeStruct((NC, NPAD, H), jnp.float32),
        mesh=_sc_mesh(),
        scratch_types=scratch,
        name="sc_scatter_add",
    )
    return f(ue, srcs, zslab)


def _node_enc_body(x_ref, mean_ref, std_ref, w0_ref, b0_ref, w1_ref, b1_ref,
                   g_ref, be_ref, o_ref):
    xn = (x_ref[...] - mean_ref[...]) / std_ref[...]
    h = jax.nn.relu(_mm(xn, w0_ref[...]) + b0_ref[...])
    h = _mm(h, w1_ref[...]) + b1_ref[...]
    o_ref[...] = _ln(h, g_ref[...], be_ref[...])


def _edge_enc_body(ea_ref, mean_ref, std_ref, w0_ref, b0_ref, w1_ref, b1_ref,
                   g_ref, be_ref, o_ref):
    ean = (ea_ref[...] - mean_ref[...]) / std_ref[...]
    h = jax.nn.relu(_mm(ean, w0_ref[...]) + b0_ref[...])
    h = _mm(h, w1_ref[...]) + b1_ref[...]
    o_ref[...] = _ln(h, g_ref[...], be_ref[...])


def _edge_mlp_body(xi_ref, xj_ref, ea_ref, wa_ref, wb_ref, wc_ref, b0_ref,
                   w1_ref, b1_ref, g_ref, be_ref, o_ref):
    ea = ea_ref[...]
    h = (_mm(xi_ref[...], wa_ref[...]) + _mm(xj_ref[...], wb_ref[...])
         + _mm(ea, wc_ref[...]) + b0_ref[...])
    h = jax.nn.relu(h)
    h = _mm(h, w1_ref[...]) + b1_ref[...]
    o_ref[...] = _ln(h, g_ref[...], be_ref[...]) + ea


def _node_mlp_body(x_ref, p0_ref, p1_ref, wa_ref, wb_ref, b0_ref,
                   w1_ref, b1_ref, g_ref, be_ref, o_ref):
    x = x_ref[...]
    agg = p0_ref[...] + p1_ref[...]
    h = _mm(x, wa_ref[...]) + _mm(agg, wb_ref[...]) + b0_ref[...]
    h = jax.nn.relu(h)
    h = _mm(h, w1_ref[...]) + b1_ref[...]
    o_ref[...] = x + _ln(h, g_ref[...], be_ref[...])


def _dec_body(x_ref, w0_ref, b0_ref, w1_ref, b1_ref, o_ref):
    h = jax.nn.relu(_mm(x_ref[...], w0_ref[...]) + b0_ref[...])
    o_ref[...] = _mm(h, w1_ref[...]) + b1_ref[...]


def _full(shape):
    nd = len(shape)
    return pl.BlockSpec(shape, lambda i: (0,) * nd)


def _rows(b, d):
    return pl.BlockSpec((b, d), lambda i: (i, 0))


def _rows3(b, d):
    return pl.BlockSpec((2, b, d), lambda i: (0, i, 0))


def _tc_call(body, nrows, brows, in_specs, out_spec, out_shape):
    return pl.pallas_call(
        body,
        grid=(nrows // brows,),
        in_specs=in_specs,
        out_specs=out_spec,
        out_shape=jax.ShapeDtypeStruct(out_shape, jnp.float32),
        compiler_params=pltpu.CompilerParams(
            dimension_semantics=("arbitrary",)),
    )


def _node_enc(x, mean_x, std_x, p):
    specs = [_rows(BN, 128), _full((128,)), _full((128,)),
             _full((128, H)), _full((H,)), _full((H, H)), _full((H,)),
             _full((H,)), _full((H,))]
    return _tc_call(_node_enc_body, NPAD, BN, specs, _rows(BN, H),
                    (NPAD, H))(
        x, mean_x, std_x, p['w0'], p['b0'], p['w1'], p['b1'], p['g'], p['be'])


def _edge_enc(ea, mean_e, std_e, p):
    specs = [_rows(BE, 4), _full((4,)), _full((4,)),
             _full((4, H)), _full((H,)), _full((H, H)), _full((H,)),
             _full((H,)), _full((H,))]
    return _tc_call(_edge_enc_body, E, BE, specs, _rows(BE, H),
                    (E, H))(
        ea, mean_e, std_e, p['w0'], p['b0'], p['w1'], p['b1'], p['g'], p['be'])


def _edge_mlp(xi, xj, ea, wa, wb, wc, p):
    specs = [_rows(BE, H), _rows(BE, H), _rows(BE, H),
             _full((H, H)), _full((H, H)), _full((H, H)), _full((H,)),
             _full((H, H)), _full((H,)), _full((H,)), _full((H,))]
    return _tc_call(_edge_mlp_body, E, BE, specs, _rows(BE, H),
                    (E, H))(
        xi, xj, ea, wa, wb, wc, p['b0'], p['w1'], p['b1'], p['g'], p['be'])


def _node_mlp(x, p0, p1, wa, wb, p):
    specs = [_rows(BN, H), _rows(BN, H), _rows(BN, H),
             _full((H, H)), _full((H, H)), _full((H,)),
             _full((H, H)), _full((H,)), _full((H,)), _full((H,))]
    return _tc_call(_node_mlp_body, NPAD, BN, specs, _rows(BN, H),
                    (NPAD, H))(
        x, p0, p1, wa, wb, p['b0'], p['w1'], p['b1'], p['g'], p['be'])


def _decoder(x, p):
    w1p = jnp.zeros((H, 8), jnp.float32).at[:, :3].set(p['w1'])
    b1p = jnp.zeros((8,), jnp.float32).at[:3].set(p['b1'])
    specs = [_rows(BN, H), _full((H, H)), _full((H,)),
             _full((H, 8)), _full((8,))]
    out = _tc_call(_dec_body, NPAD, BN, specs, _rows(BN, 8), (NPAD, 8))(
        x, p['w0'], p['b0'], w1p, b1p)
    return out[:N, :3]


def kernel(x, edge_index, edge_attr, mean_x, std_x, mean_edge, std_edge,
           params):
    src = edge_index[0]
    dst = edge_index[1]
    src2 = src.reshape(NS, NCH, CH)
    dst2 = dst.reshape(NS, NCH, CH)
    srcs = src.reshape(NW, NCHS, CHS)

    xp = jnp.zeros((NPAD, 128), jnp.float32).at[:N].set(x)
    x = _node_enc(xp, mean_x, std_x, params['node_enc'])
    ea = _edge_enc(edge_attr, mean_edge, std_edge, params['edge_enc'])

    zslab = jnp.zeros((NSLAB, H), jnp.float32)
    for lp in params['layers']:
        ew = lp['edge']
        wa, wb, wc = ew['w0'][:H], ew['w0'][H:2 * H], ew['w0'][2 * H:]
        xi, xj = _sc_gather2(x, dst2, src2)
        ue = _edge_mlp(xi, xj, ea, wa, wb, wc, ew)
        part = _sc_scatter_add(ue, srcs, zslab)
        nw = lp['node']
        x = _node_mlp(x, part[0], part[1], nw['w0'][:H], nw['w0'][H:], nw)
        ea = ue

    return _decoder(x, params['dec'])
